# Initial kernel scaffold; baseline (speedup 1.0000x reference)
import jax
import jax.numpy as jnp
from jax.experimental import pallas as pl


def kernel(x, Wr, W1, b1, W2, b2):
    raise NotImplementedError("write your pallas kernel here")


# speedup vs baseline: 1.9523x; 1.9523x over previous
import functools

import jax
import jax.numpy as jnp
from jax.experimental import pallas as pl
from jax.experimental.pallas import tpu as pltpu

E = 8
TOP_K = 2
T = 2048
C = 768
H = 1024


def _moe_dense_body(x_ref, Wr_ref, W1_ref, b1_ref, W2_ref, b2_ref,
                    out_ref, gates_ref):
    e = pl.program_id(0)
    eidx = jax.lax.broadcasted_iota(jnp.int32, (T, E), 1)

    @pl.when(e == 0)
    def _():
        logits = jax.lax.dot_general(
            x_ref[...], Wr_ref[...], (((1,), (1,)), ((), ())),
            preferred_element_type=jnp.float32)
        m1 = jnp.max(logits, axis=1, keepdims=True)
        am1 = jnp.min(jnp.where(logits == m1, eidx, E), axis=1, keepdims=True)
        masked = jnp.where(eidx == am1, -jnp.inf, logits)
        m2 = jnp.max(masked, axis=1, keepdims=True)
        am2 = jnp.min(jnp.where(masked == m2, eidx, E), axis=1, keepdims=True)
        g1 = 1.0 / (1.0 + jnp.exp(m2 - m1))
        g2 = 1.0 - g1
        gates_ref[...] = (jnp.where(eidx == am1, g1, 0.0)
                          + jnp.where(eidx == am2, g2, 0.0))
        out_ref[...] = jnp.zeros_like(out_ref)

    h = jax.lax.dot_general(
        x_ref[...], W1_ref[0], (((1,), (1,)), ((), ())),
        preferred_element_type=jnp.float32) + b1_ref[0]
    h = jnp.maximum(h, 0.0)
    y = jax.lax.dot_general(
        h, W2_ref[0], (((1,), (1,)), ((), ())),
        preferred_element_type=jnp.float32) + b2_ref[0]
    ge = jnp.sum(jnp.where(eidx == e, gates_ref[...], 0.0), axis=1,
                 keepdims=True)
    out_ref[...] += y * ge


@jax.jit
def _moe_dense(x2d, Wr, W1, b1, W2, b2):
    return pl.pallas_call(
        _moe_dense_body,
        grid=(E,),
        in_specs=[
            pl.BlockSpec((T, C), lambda e: (0, 0)),
            pl.BlockSpec((E, C), lambda e: (0, 0)),
            pl.BlockSpec((1, H, C), lambda e: (e, 0, 0)),
            pl.BlockSpec((1, 1, H), lambda e: (e, 0, 0)),
            pl.BlockSpec((1, C, H), lambda e: (e, 0, 0)),
            pl.BlockSpec((1, 1, C), lambda e: (e, 0, 0)),
        ],
        out_specs=pl.BlockSpec((T, C), lambda e: (0, 0)),
        out_shape=jax.ShapeDtypeStruct((T, C), jnp.float32),
        scratch_shapes=[pltpu.VMEM((T, E), jnp.float32)],
        compiler_params=pltpu.CompilerParams(
            dimension_semantics=("arbitrary",)),
    )(x2d, Wr, W1, b1.reshape(E, 1, H), W2, b2.reshape(E, 1, C))


def kernel(x, Wr, W1, b1, W2, b2):
    Bs, Ts, Cs = x.shape
    out = _moe_dense(x.reshape(Ts, Cs), Wr, W1, b1, W2, b2)
    return out.reshape(Bs, Ts, Cs)
